# baseline (speedup 1.0000x reference)
import jax
import jax.numpy as jnp
from jax.experimental import pallas as pl
from jax.experimental.pallas import tpu as pltpu

B = 8
N = 2048
KC = 256
IC = 512


def _fused_body(srow_ref, scol_ref, out_ref, lo_ref, hi_ref, valid_ref):
    t = pl.program_id(1)

    @pl.when(t == 0)
    def _():
        srow = srow_ref[0]
        acc = None
        bacc = None
        for c in range(N // KC):
            sk = scol_ref[0, c * KC:(c + 1) * KC, :]
            gt = sk > srow
            eq = sk == srow
            kio = jax.lax.broadcasted_iota(jnp.int32, (KC, N), 0) + c * KC
            jio = jax.lax.broadcasted_iota(jnp.int32, (KC, N), 1)
            cnt = jnp.where(gt, 1, 0) + jnp.where(eq, 65536, 0)
            bc = jnp.where(eq & (kio < jio), 1, 0)
            acc = cnt if acc is None else acc + cnt
            bacc = bc if bacc is None else bacc + bc
        tot = jnp.sum(acc, axis=0, keepdims=True)
        before = jnp.sum(bacc, axis=0, keepdims=True)
        lo = tot & 65535
        lo_ref[...] = lo
        hi_ref[...] = lo + (tot >> 16)
        valid_ref[...] = jnp.where(before == 0, 1, 0)

    lo = lo_ref[...]
    hi = hi_ref[...]
    valid = valid_ref[...] == 1
    iio = jax.lax.broadcasted_iota(jnp.int32, (IC, N), 0) + t * IC
    p = (iio >= lo) & (iio < hi) & valid
    out_ref[0] = p.astype(jnp.float32)


def kernel(s):
    return pl.pallas_call(
        _fused_body,
        grid=(B, N // IC),
        in_specs=[
            pl.BlockSpec((1, 1, N), lambda b, t: (b, 0, 0)),
            pl.BlockSpec((1, N, 1), lambda b, t: (b, 0, 0)),
        ],
        out_specs=pl.BlockSpec((1, IC, N), lambda b, t: (b, t, 0)),
        out_shape=jax.ShapeDtypeStruct((B, N, N), jnp.float32),
        scratch_shapes=[
            pltpu.VMEM((1, N), jnp.int32),
            pltpu.VMEM((1, N), jnp.int32),
            pltpu.VMEM((1, N), jnp.int32),
        ],
    )(s.reshape(B, 1, N), s.reshape(B, N, 1))
